# Initial kernel scaffold; baseline (speedup 1.0000x reference)
import functools

import jax
import jax.numpy as jnp
from jax import lax
from jax.experimental import pallas as pl
from jax.experimental.pallas import tpu as pltpu
from jax.experimental.pallas import tpu_sc as plsc

B, T, V = 1024, 50, 1000
N = B * T
NC, NS = 2, 16
NW = NC * NS
PER_W = N // NW
C = 64
NCH = PER_W // C


def _lse_body(emb_ref, out_ref):
    x = emb_ref[...]
    m = jnp.max(x, axis=1, keepdims=True)
    s = jnp.sum(jnp.exp(x - m), axis=1, keepdims=True)
    out_ref[...] = m + jnp.log(s)


def _mean_body(nll_ref, out_ref):
    out_ref[...] = (jnp.sum(nll_ref[...]) / N).reshape(1, 1)


def _sc_body(emb_hbm, embf_hbm, lse_hbm, idx_hbm, tgt_hbm,
             logits_hbm, nll_hbm,
             rows_v, idx_v, tgt_v, fidx_v, tv_v, lseg_v, nll_v,
             rsem, tsem, lsem):
    wid = lax.axis_index("s") * NC + lax.axis_index("c")
    wbase = wid * PER_W

    def chunk_body(ci, carry):
        base = wbase + ci * C
        pltpu.sync_copy(idx_hbm.at[pl.ds(base, C)], idx_v)
        pltpu.sync_copy(tgt_hbm.at[pl.ds(base, C)], tgt_v)
        rcopy = pltpu.async_copy(emb_hbm.at[idx_v], rows_v, rsem)
        for j in range(C // 16):
            sl = pl.ds(j * 16, 16)
            fidx_v[sl] = idx_v[sl] * V + tgt_v[sl]
        tcopy = pltpu.async_copy(embf_hbm.at[fidx_v], tv_v, tsem)
        lcopy = pltpu.async_copy(lse_hbm.at[idx_v], lseg_v, lsem)
        tcopy.wait()
        lcopy.wait()
        for j in range(C // 16):
            sl = pl.ds(j * 16, 16)
            nll_v[sl] = lseg_v[sl] - tv_v[sl]
        pltpu.sync_copy(nll_v, nll_hbm.at[pl.ds(base, C)])
        rcopy.wait()
        pltpu.sync_copy(rows_v, logits_hbm.at[pl.ds(base, C)])
        return carry

    lax.fori_loop(0, NCH, chunk_body, None)


def kernel(idx, targets, emb):
    lse2 = pl.pallas_call(
        _lse_body,
        out_shape=jax.ShapeDtypeStruct((V, 1), jnp.float32),
    )(emb)
    lse = lse2.reshape(V)

    mesh = plsc.VectorSubcoreMesh(core_axis_name="c", subcore_axis_name="s")
    sc = functools.partial(
        pl.kernel,
        mesh=mesh,
        out_type=(
            jax.ShapeDtypeStruct((N, V), jnp.float32),
            jax.ShapeDtypeStruct((N,), jnp.float32),
        ),
        scratch_types=[
            pltpu.VMEM((C, V), jnp.float32),
            pltpu.VMEM((C,), jnp.int32),
            pltpu.VMEM((C,), jnp.int32),
            pltpu.VMEM((C,), jnp.int32),
            pltpu.VMEM((C,), jnp.float32),
            pltpu.VMEM((C,), jnp.float32),
            pltpu.VMEM((C,), jnp.float32),
            pltpu.SemaphoreType.DMA,
            pltpu.SemaphoreType.DMA,
            pltpu.SemaphoreType.DMA,
        ],
    )(_sc_body)
    logits2, nll = sc(emb, emb.reshape(-1), lse,
                      idx.reshape(-1), targets.reshape(-1))

    loss2 = pl.pallas_call(
        _mean_body,
        out_shape=jax.ShapeDtypeStruct((1, 1), jnp.float32),
    )(nll.reshape(N // 128, 128))
    return (logits2, loss2[0, 0])


# speedup vs baseline: 1.5861x; 1.5861x over previous
import functools

import jax
import jax.numpy as jnp
from jax import lax
from jax.experimental import pallas as pl
from jax.experimental.pallas import tpu as pltpu
from jax.experimental.pallas import tpu_sc as plsc

B, T, V = 1024, 50, 1000
N = B * T
NC, NS = 2, 16
NW = NC * NS
PER_W = N // NW
C = 64
NCH = PER_W // C
VV = V * V
NOUT = 8
CHUNK_OUT = VV // NOUT


def _loss_body(emb_ref, c2_ref, out_ref):
    x = emb_ref[...]
    m = jnp.max(x, axis=1, keepdims=True)
    lse = m + jnp.log(jnp.sum(jnp.exp(x - m), axis=1, keepdims=True))
    w = c2_ref[0:V, :] + c2_ref[V:2 * V, :]
    out_ref[...] = (jnp.sum(w * (lse - x)) / N).reshape(1, 1)


def _sc_body(emb_hbm, idx_hbm, tgt_hbm, zeros_hbm,
             logits_hbm, c2_hbm,
             rows_v, idx_v, tgt_v, fidx_v, ones_v,
             shared, rsem):
    cid = lax.axis_index("c")
    sid = lax.axis_index("s")
    wid = sid * NC + cid
    wbase = wid * PER_W

    @pl.when(sid == 0)
    def _():
        pltpu.sync_copy(zeros_hbm, shared)

    for j in range(C // 16):
        ones_v[pl.ds(j * 16, 16)] = jnp.full((16,), 1.0, jnp.float32)
    plsc.subcore_barrier()

    def chunk_body(ci, carry):
        base = wbase + ci * C
        pltpu.sync_copy(idx_hbm.at[pl.ds(base, C)], idx_v)
        pltpu.sync_copy(tgt_hbm.at[pl.ds(base, C)], tgt_v)
        rcopy = pltpu.async_copy(emb_hbm.at[idx_v], rows_v, rsem)
        for j in range(C // 16):
            sl = pl.ds(j * 16, 16)
            fidx_v[sl] = idx_v[sl] * V + tgt_v[sl]
        pltpu.sync_copy(ones_v, shared.at[fidx_v], add=True)
        rcopy.wait()
        pltpu.sync_copy(rows_v, logits_hbm.at[pl.ds(base, C)])
        return carry

    lax.fori_loop(0, NCH, chunk_body, None)

    plsc.subcore_barrier()

    @pl.when(sid < NOUT)
    def _():
        pltpu.sync_copy(shared.at[pl.ds(sid * CHUNK_OUT, CHUNK_OUT)],
                        c2_hbm.at[cid, pl.ds(sid * CHUNK_OUT, CHUNK_OUT)])


def kernel(idx, targets, emb):
    mesh = plsc.VectorSubcoreMesh(core_axis_name="c", subcore_axis_name="s")
    sc = functools.partial(
        pl.kernel,
        mesh=mesh,
        compiler_params=pltpu.CompilerParams(use_tc_tiling_on_sc=False),
        out_type=(
            jax.ShapeDtypeStruct((N, V), jnp.float32),
            jax.ShapeDtypeStruct((NC, VV), jnp.float32),
        ),
        scratch_types=[
            pltpu.VMEM((C, V), jnp.float32),
            pltpu.VMEM((C,), jnp.int32),
            pltpu.VMEM((C,), jnp.int32),
            pltpu.VMEM((C,), jnp.int32),
            pltpu.VMEM((C,), jnp.float32),
            pltpu.VMEM_SHARED((VV,), jnp.float32),
            pltpu.SemaphoreType.DMA,
        ],
    )(_sc_body)
    zeros = jnp.zeros((VV,), jnp.float32)
    logits2, c2 = sc(emb, idx.reshape(-1), targets.reshape(-1), zeros)

    loss2 = pl.pallas_call(
        _loss_body,
        out_shape=jax.ShapeDtypeStruct((1, 1), jnp.float32),
    )(emb, c2.reshape(NC * V, V))
    return (logits2, loss2[0, 0])


# speedup vs baseline: 1.6607x; 1.0470x over previous
import functools

import jax
import jax.numpy as jnp
from jax import lax
from jax.experimental import pallas as pl
from jax.experimental.pallas import tpu as pltpu
from jax.experimental.pallas import tpu_sc as plsc

B, T, V = 1024, 50, 1000
N = B * T
NC, NS = 2, 16
NW = NC * NS
PER_W = N // NW
C = 32
NCH = PER_W // C
VV = V * V
NOUT = 8
CHUNK_OUT = VV // NOUT


def _loss_body(emb_ref, c2_ref, out_ref):
    x = emb_ref[...]
    m = jnp.max(x, axis=1, keepdims=True)
    lse = m + jnp.log(jnp.sum(jnp.exp(x - m), axis=1, keepdims=True))
    w = c2_ref[0:V, :] + c2_ref[V:2 * V, :]
    out_ref[...] = (jnp.sum(w * (lse - x)) / N).reshape(1, 1)


def _sc_body(emb_hbm, idx_hbm, tgt_hbm, zeros_hbm,
             logits_hbm, c2_hbm,
             rows0, rows1, idx_all, tgt_all, idxv0, idxv1, fidx_v, ones_v,
             shared, sem0, sem1):
    cid = lax.axis_index("c")
    sid = lax.axis_index("s")
    wid = sid * NC + cid
    wbase = wid * PER_W

    @pl.when(sid == 0)
    def _():
        pltpu.sync_copy(zeros_hbm, shared)

    pltpu.sync_copy(idx_hbm.at[pl.ds(wbase, PER_W)], idx_all)
    pltpu.sync_copy(tgt_hbm.at[pl.ds(wbase, PER_W)], tgt_all)
    for j in range(C // 16):
        ones_v[pl.ds(j * 16, 16)] = jnp.full((16,), 1.0, jnp.float32)
    plsc.subcore_barrier()

    rows = (rows0, rows1)
    idxv = (idxv0, idxv1)
    sems = (sem0, sem1)

    def fill_and_fire(b, ci):
        for j in range(C // 16):
            sl = pl.ds(j * 16, 16)
            idxv[b][sl] = idx_all[pl.ds(ci * C + j * 16, 16)]
        pltpu.async_copy(emb_hbm.at[idxv[b]], rows[b], sems[b])

    def hist(ci):
        for j in range(C // 16):
            sl = pl.ds(j * 16, 16)
            fidx_v[sl] = (idx_all[pl.ds(ci * C + j * 16, 16)] * V
                          + tgt_all[pl.ds(ci * C + j * 16, 16)])
        pltpu.sync_copy(ones_v, shared.at[fidx_v], add=True)

    def drain_and_out(b, ci):
        pltpu.make_async_copy(emb_hbm.at[idxv[b]], rows[b], sems[b]).wait()
        pltpu.sync_copy(rows[b], logits_hbm.at[pl.ds(wbase + ci * C, C)])

    fill_and_fire(0, 0)

    def pair_body(i, carry):
        fill_and_fire(1, i + 1)
        hist(i)
        drain_and_out(0, i)

        @pl.when(i + 2 < NCH)
        def _():
            fill_and_fire(0, i + 2)

        hist(i + 1)
        drain_and_out(1, i + 1)
        return carry

    lax.fori_loop(0, NCH // 2, lambda k, c: pair_body(k * 2, c), None)

    plsc.subcore_barrier()

    @pl.when(sid < NOUT)
    def _():
        pltpu.sync_copy(shared.at[pl.ds(sid * CHUNK_OUT, CHUNK_OUT)],
                        c2_hbm.at[cid, pl.ds(sid * CHUNK_OUT, CHUNK_OUT)])


def kernel(idx, targets, emb):
    mesh = plsc.VectorSubcoreMesh(core_axis_name="c", subcore_axis_name="s")
    sc = functools.partial(
        pl.kernel,
        mesh=mesh,
        compiler_params=pltpu.CompilerParams(use_tc_tiling_on_sc=False),
        out_type=(
            jax.ShapeDtypeStruct((N, V), jnp.float32),
            jax.ShapeDtypeStruct((NC, VV), jnp.float32),
        ),
        scratch_types=[
            pltpu.VMEM((C, V), jnp.float32),
            pltpu.VMEM((C, V), jnp.float32),
            pltpu.VMEM((PER_W,), jnp.int32),
            pltpu.VMEM((PER_W,), jnp.int32),
            pltpu.VMEM((C,), jnp.int32),
            pltpu.VMEM((C,), jnp.int32),
            pltpu.VMEM((C,), jnp.int32),
            pltpu.VMEM((C,), jnp.float32),
            pltpu.VMEM_SHARED((VV,), jnp.float32),
            pltpu.SemaphoreType.DMA,
            pltpu.SemaphoreType.DMA,
        ],
    )(_sc_body)
    zeros = jnp.zeros((VV,), jnp.float32)
    logits2, c2 = sc(emb, idx.reshape(-1), targets.reshape(-1), zeros)

    loss2 = pl.pallas_call(
        _loss_body,
        out_shape=jax.ShapeDtypeStruct((1, 1), jnp.float32),
    )(emb, c2.reshape(NC * V, V))
    return (logits2, loss2[0, 0])
